# baseline (speedup 1.0000x reference)
import functools
import jax
import jax.numpy as jnp
from jax import lax
from jax.experimental import pallas as pl
from jax.experimental.pallas import tpu as pltpu
from jax.experimental.pallas import tpu_sc as plsc

B = 4
H = W_IN = 112
C = 96
HOUT = 224
FLAT_OUT = HOUT * HOUT * C
N_IN = H * W_IN * C
K_CHUNKS = 4
CH = FLAT_OUT // K_CHUNKS
JUNK = 8192
ACC_LEN = CH + JUNK
N_TILES = 16
E_T = N_IN // N_TILES
WIN = 3136
N_WIN = E_T // WIN
NBUF = 4
TILE_CH = CH // N_TILES
ZCOPY = 4704
N_ZCOPY = TILE_CH // ZCOPY
DCOPY = 4704
N_DCOPY = TILE_CH // DCOPY
NDBUF = 4
TOTAL_OUT = B * FLAT_OUT


def _unpool_body(vals_hbm, idx_hbm, out_hbm,
                 i0, i1, i2, i3, v0, v1, v2, v3,
                 zbuf, d0, d1, d2, d3, acc, sem_l, sem_s, sem_z,
                 sem_dr, sem_dw):
    dbuf = (d0, d1, d2, d3)
    idx_buf = (i0, i1, i2, i3)
    val_buf = (v0, v1, v2, v3)
    c = lax.axis_index("c")
    sid = lax.axis_index("s")
    lanes = lax.iota(jnp.int32, 16)
    junk_vec = (jnp.int32(CH)
                + (sid.astype(jnp.int32) * 16 + lanes) * 32)
    zeros16 = jnp.zeros((16,), jnp.float32)

    def zb_body(i, _):
        zbuf[pl.ds(i * 16, 16)] = zeros16
        return 0

    lax.fori_loop(0, ZCOPY // 16, zb_body, 0)

    def start_loads(bf, off):
        ci = pltpu.async_copy(idx_hbm.at[pl.ds(off, WIN)], idx_buf[bf],
                              sem_l.at[bf])
        cv = pltpu.async_copy(vals_hbm.at[pl.ds(off, WIN)], val_buf[bf],
                              sem_l.at[bf])
        return (ci, cv)

    zcps = [pltpu.async_copy(
        zbuf, acc.at[pl.ds(sid * TILE_CH + i * ZCOPY, ZCOPY)],
        sem_z) for i in range(N_ZCOPY)]
    for cp in zcps:
        cp.wait()
    plsc.subcore_barrier()

    @pl.loop(0, 2)
    def _(bb):
        b = c * 2 + bb

        @pl.loop(0, K_CHUNKS)
        def _(k):
            lo = k * CH
            in_base = b * N_IN + sid * E_T
            out_base = b * FLAT_OUT + lo

            loads = [None] * N_WIN
            scats = [None] * N_WIN
            for v in range(2):
                loads[v] = start_loads(v, in_base + v * WIN)
            for v in range(N_WIN):
                bf = v % NBUF
                if v >= 2:
                    scats[v - 2].wait()
                if v + 2 < N_WIN:
                    loads[v + 2] = start_loads((v + 2) % NBUF,
                                               in_base + (v + 2) * WIN)
                loads[v][0].wait()
                loads[v][1].wait()

                def tf_body(i, _):
                    base = i * 112
                    for r in range(7):
                        t = idx_buf[bf][pl.ds(base + r * 16, 16)]
                        u = t - lo
                        in_range = (plsc.bitcast(u, jnp.uint32)
                                    < jnp.uint32(CH))
                        idx_buf[bf][pl.ds(base + r * 16, 16)] = (
                            jnp.where(in_range, u, junk_vec))
                    return 0

                lax.fori_loop(0, WIN // 112, tf_body, 0)
                scats[v] = pltpu.async_copy(val_buf[bf], acc.at[idx_buf[bf]],
                                            sem_s.at[bf], add=True)
            scats[N_WIN - 2].wait()
            scats[N_WIN - 1].wait()
            plsc.subcore_barrier()

            def acc_blk(i):
                return acc.at[pl.ds(sid * TILE_CH + i * DCOPY, DCOPY)]

            def hbm_blk(i):
                return out_hbm.at[pl.ds(out_base + sid * TILE_CH + i * DCOPY,
                                        DCOPY)]

            dreads = [None] * N_DCOPY
            dwrites = [None] * N_DCOPY
            zcps = [None] * N_DCOPY
            for i in range(2):
                dreads[i] = pltpu.async_copy(acc_blk(i), dbuf[i], sem_dr.at[i])
            for i in range(N_DCOPY):
                db = i % NDBUF
                if i >= 2:
                    dwrites[i - 2].wait()
                if i + 2 < N_DCOPY:
                    dreads[i + 2] = pltpu.async_copy(
                        acc_blk(i + 2), dbuf[(i + 2) % NDBUF],
                        sem_dr.at[(i + 2) % NDBUF])
                dreads[i].wait()
                dwrites[i] = pltpu.async_copy(dbuf[db], hbm_blk(i),
                                              sem_dw.at[db])
                zcps[i] = pltpu.async_copy(
                    zbuf, acc.at[pl.ds(sid * TILE_CH + i * DCOPY, DCOPY)],
                    sem_z)
            dwrites[N_DCOPY - 2].wait()
            dwrites[N_DCOPY - 1].wait()
            for cp in zcps:
                cp.wait()
            plsc.subcore_barrier()


@jax.jit
def _unpool(vals, idx):
    mesh = plsc.VectorSubcoreMesh(core_axis_name="c", subcore_axis_name="s")
    f = functools.partial(
        pl.kernel,
        mesh=mesh,
        out_type=jax.ShapeDtypeStruct((TOTAL_OUT,), jnp.float32),
        scratch_types=(
            [pltpu.VMEM((WIN,), jnp.int32) for _ in range(NBUF)]
            + [pltpu.VMEM((WIN,), jnp.float32) for _ in range(NBUF)]
            + [pltpu.VMEM((ZCOPY,), jnp.float32)]
            + [pltpu.VMEM((DCOPY,), jnp.float32) for _ in range(4)]
            + [pltpu.VMEM_SHARED((ACC_LEN,), jnp.float32),
               pltpu.SemaphoreType.DMA((NBUF,)),
               pltpu.SemaphoreType.DMA((NBUF,)),
               pltpu.SemaphoreType.DMA,
               pltpu.SemaphoreType.DMA((4,)),
               pltpu.SemaphoreType.DMA((4,))]
        ),
    )(_unpool_body)
    return f(vals, idx)


def kernel(inputs, argmax, spatial_output_shape):
    del spatial_output_shape
    vals = inputs.reshape(-1)
    idx = argmax.astype(jnp.int32).reshape(-1)
    out = _unpool(vals, idx)
    return out.reshape(B, HOUT, HOUT, C)


# speedup vs baseline: 1.7643x; 1.7643x over previous
import functools
import jax
import jax.numpy as jnp
from jax import lax
from jax.experimental import pallas as pl
from jax.experimental.pallas import tpu as pltpu
from jax.experimental.pallas import tpu_sc as plsc

B = 4
H = W_IN = 112
C = 96
HOUT = 224
FLAT_OUT = HOUT * HOUT * C
N_IN = H * W_IN * C
K_CHUNKS = 4
CH = FLAT_OUT // K_CHUNKS
JUNK = 256
ACC_LEN = CH + JUNK
N_TILES = 16
E_T = N_IN // N_TILES
WIN = 3136
N_WIN = E_T // WIN
NBUF = 4
TILE_CH = CH // N_TILES
ZCOPY = 4704
N_ZCOPY = TILE_CH // ZCOPY
DCOPY = 4704
N_DCOPY = TILE_CH // DCOPY
NDBUF = 4
TOTAL_OUT = B * FLAT_OUT


def _unpool_body(vals_hbm, idx_hbm, out_hbm,
                 i0, i1, i2, i3, v0, v1, v2, v3,
                 zbuf, d0, d1, d2, d3, acc, sem_l, sem_s, sem_z,
                 sem_dr, sem_dw):
    dbuf = (d0, d1, d2, d3)
    idx_buf = (i0, i1, i2, i3)
    val_buf = (v0, v1, v2, v3)
    c = lax.axis_index("c")
    sid = lax.axis_index("s")
    lanes = lax.iota(jnp.int32, 16)
    junk_vec = jnp.int32(CH) + sid.astype(jnp.int32) * 16 + lanes
    zeros16 = jnp.zeros((16,), jnp.float32)

    def zb_body(i, _):
        zbuf[pl.ds(i * 16, 16)] = zeros16
        return 0

    lax.fori_loop(0, ZCOPY // 16, zb_body, 0)

    def start_loads(bf, off):
        ci = pltpu.async_copy(idx_hbm.at[pl.ds(off, WIN)], idx_buf[bf],
                              sem_l.at[bf])
        cv = pltpu.async_copy(vals_hbm.at[pl.ds(off, WIN)], val_buf[bf],
                              sem_l.at[bf])
        return (ci, cv)

    zcps = [pltpu.async_copy(
        zbuf, acc.at[pl.ds(sid * TILE_CH + i * ZCOPY, ZCOPY)],
        sem_z) for i in range(N_ZCOPY)]
    for cp in zcps:
        cp.wait()
    plsc.subcore_barrier()

    @pl.loop(0, 2)
    def _(bb):
        b = c * 2 + bb

        @pl.loop(0, K_CHUNKS)
        def _(k):
            lo = k * CH
            in_base = b * N_IN + sid * E_T
            out_base = b * FLAT_OUT + lo

            loads = [None] * N_WIN
            scats = [None] * N_WIN
            for v in range(2):
                loads[v] = start_loads(v, in_base + v * WIN)
            for v in range(N_WIN):
                bf = v % NBUF
                if v >= 2:
                    scats[v - 2].wait()
                if v + 2 < N_WIN:
                    loads[v + 2] = start_loads((v + 2) % NBUF,
                                               in_base + (v + 2) * WIN)
                loads[v][0].wait()
                loads[v][1].wait()

                def tf_body(i, _):
                    base = i * 112
                    for r in range(7):
                        t = idx_buf[bf][pl.ds(base + r * 16, 16)]
                        u = t - lo
                        in_range = (plsc.bitcast(u, jnp.uint32)
                                    < jnp.uint32(CH))
                        idx_buf[bf][pl.ds(base + r * 16, 16)] = (
                            jnp.where(in_range, u, junk_vec))
                    return 0

                lax.fori_loop(0, WIN // 112, tf_body, 0)
                scats[v] = pltpu.async_copy(val_buf[bf], acc.at[idx_buf[bf]],
                                            sem_s.at[bf], add=True)
            scats[N_WIN - 2].wait()
            scats[N_WIN - 1].wait()
            plsc.subcore_barrier()

            def acc_blk(i):
                return acc.at[pl.ds(sid * TILE_CH + i * DCOPY, DCOPY)]

            def hbm_blk(i):
                return out_hbm.at[pl.ds(out_base + sid * TILE_CH + i * DCOPY,
                                        DCOPY)]

            dreads = [None] * N_DCOPY
            dwrites = [None] * N_DCOPY
            zcps = [None] * N_DCOPY
            for i in range(2):
                dreads[i] = pltpu.async_copy(acc_blk(i), dbuf[i], sem_dr.at[i])
            for i in range(N_DCOPY):
                db = i % NDBUF
                if i >= 2:
                    dwrites[i - 2].wait()
                if i + 2 < N_DCOPY:
                    dreads[i + 2] = pltpu.async_copy(
                        acc_blk(i + 2), dbuf[(i + 2) % NDBUF],
                        sem_dr.at[(i + 2) % NDBUF])
                dreads[i].wait()
                dwrites[i] = pltpu.async_copy(dbuf[db], hbm_blk(i),
                                              sem_dw.at[db])
                zcps[i] = pltpu.async_copy(
                    zbuf, acc.at[pl.ds(sid * TILE_CH + i * DCOPY, DCOPY)],
                    sem_z)
            dwrites[N_DCOPY - 2].wait()
            dwrites[N_DCOPY - 1].wait()
            for cp in zcps:
                cp.wait()
            plsc.subcore_barrier()


@jax.jit
def _unpool(vals, idx):
    mesh = plsc.VectorSubcoreMesh(core_axis_name="c", subcore_axis_name="s")
    f = functools.partial(
        pl.kernel,
        mesh=mesh,
        out_type=jax.ShapeDtypeStruct((TOTAL_OUT,), jnp.float32),
        scratch_types=(
            [pltpu.VMEM((WIN,), jnp.int32) for _ in range(NBUF)]
            + [pltpu.VMEM((WIN,), jnp.float32) for _ in range(NBUF)]
            + [pltpu.VMEM((ZCOPY,), jnp.float32)]
            + [pltpu.VMEM((DCOPY,), jnp.float32) for _ in range(4)]
            + [pltpu.VMEM_SHARED((ACC_LEN,), jnp.float32),
               pltpu.SemaphoreType.DMA((NBUF,)),
               pltpu.SemaphoreType.DMA((NBUF,)),
               pltpu.SemaphoreType.DMA,
               pltpu.SemaphoreType.DMA((4,)),
               pltpu.SemaphoreType.DMA((4,))]
        ),
    )(_unpool_body)
    return f(vals, idx)


def kernel(inputs, argmax, spatial_output_shape):
    del spatial_output_shape
    vals = inputs.reshape(-1)
    idx = argmax.astype(jnp.int32).reshape(-1)
    out = _unpool(vals, idx)
    return out.reshape(B, HOUT, HOUT, C)
